# baseline (speedup 1.0000x reference)
import functools

import jax
import jax.numpy as jnp
from jax import lax
from jax.experimental import pallas as pl
from jax.experimental.pallas import tpu as pltpu
from jax.experimental.pallas import tpu_sc as plsc

VOCAB = 23236
EMBED_DIM = 1728
BATCH = 32
SEQ = 512

_NC = 2
_NS = 16
_NW = _NC * _NS

_B = BATCH * SEQ
_BPW = _B // _NW
_K = 32
_NCH = _BPW // _K


def _gather_body(table_hbm, ids_hbm, out_hbm, idx_v, rows0, rows1,
                 gsem0, gsem1):
    wid = lax.axis_index("s") * _NC + lax.axis_index("c")
    base = wid * _BPW

    pltpu.sync_copy(ids_hbm.at[pl.ds(base, _BPW)], idx_v)

    def start_gather(c, rows, sem):
        return pltpu.async_copy(
            table_hbm.at[idx_v.at[pl.ds(c * _K, _K)]], rows, sem)

    start_gather(0, rows0, gsem0)
    start_gather(1, rows1, gsem1)

    def step(c, rows, sem):
        pltpu.make_async_copy(
            table_hbm.at[idx_v.at[pl.ds(c * _K, _K)]], rows, sem).wait()
        pltpu.sync_copy(rows, out_hbm.at[pl.ds(base + c * _K, _K)])

        @pl.when(c + 2 < _NCH)
        def _():
            start_gather(c + 2, rows, sem)

    def pair(i, _):
        step(2 * i, rows0, gsem0)
        step(2 * i + 1, rows1, gsem1)
        return _

    lax.fori_loop(0, _NCH // 2, pair, 0)


@jax.jit
def _embed(ids_flat, font_table):
    mesh = plsc.VectorSubcoreMesh(core_axis_name="c", subcore_axis_name="s")
    run = pl.kernel(
        _gather_body,
        out_type=jax.ShapeDtypeStruct((_B, EMBED_DIM), jnp.float32),
        mesh=mesh,
        scratch_types=[
            pltpu.VMEM((_BPW,), jnp.int32),
            pltpu.VMEM((_K, EMBED_DIM), jnp.float32),
            pltpu.VMEM((_K, EMBED_DIM), jnp.float32),
            pltpu.SemaphoreType.DMA,
            pltpu.SemaphoreType.DMA,
        ],
        compiler_params=pltpu.CompilerParams(use_tc_tiling_on_sc=False),
    )
    return run(font_table, ids_flat)


def kernel(input_ids, font_table):
    ids_flat = input_ids.reshape(-1).astype(jnp.int32)
    out = _embed(ids_flat, font_table)
    return out.reshape(BATCH, SEQ, EMBED_DIM)


# speedup vs baseline: 3.3385x; 3.3385x over previous
import functools

import jax
import jax.numpy as jnp
from jax import lax
from jax.experimental import pallas as pl
from jax.experimental.pallas import tpu as pltpu
from jax.experimental.pallas import tpu_sc as plsc

VOCAB = 23236
EMBED_DIM = 1728
BATCH = 32
SEQ = 512

_NC = 2
_NS = 16
_NW = _NC * _NS

_B = BATCH * SEQ
_BPW = _B // _NW
_K = 32
_NCH = _BPW // _K
_NFULL = EMBED_DIM // 128
_TAIL0 = _NFULL * 128


def _gather_body(table_hbm, tail_hbm, ids_hbm, out_hbm, idx_v,
                 rows0, rows1, tail0, tail1, gsem0, gsem1):
    wid = lax.axis_index("s") * _NC + lax.axis_index("c")
    base = wid * _BPW

    pltpu.sync_copy(ids_hbm.at[pl.ds(base, _BPW)], idx_v)

    def start_gathers(c, rows, tail, sem):
        idx = idx_v.at[pl.ds(c * _K, _K)]
        for j in range(_NFULL):
            pltpu.async_copy(
                table_hbm.at[idx, pl.ds(j * 128, 128)],
                rows.at[:, pl.ds(j * 128, 128)], sem)
        pltpu.async_copy(tail_hbm.at[idx], tail, sem)

    def wait_gathers(c, rows, tail, sem):
        idx = idx_v.at[pl.ds(c * _K, _K)]
        for j in range(_NFULL):
            pltpu.make_async_copy(
                table_hbm.at[idx, pl.ds(j * 128, 128)],
                rows.at[:, pl.ds(j * 128, 128)], sem).wait()
        pltpu.make_async_copy(tail_hbm.at[idx], tail, sem).wait()

    start_gathers(0, rows0, tail0, gsem0)
    start_gathers(1, rows1, tail1, gsem1)

    def step(c, rows, tail, sem):
        wait_gathers(c, rows, tail, sem)

        def merge_row(r, _):
            for k in range(4):
                rows[r, pl.ds(_TAIL0 + 16 * k, 16)] = \
                    tail[r, pl.ds(16 * k, 16)]
            return _
        lax.fori_loop(0, _K, merge_row, 0)

        pltpu.sync_copy(rows, out_hbm.at[pl.ds(base + c * _K, _K)])

        @pl.when(c + 2 < _NCH)
        def _():
            start_gathers(c + 2, rows, tail, sem)

    def pair(i, _):
        step(2 * i, rows0, tail0, gsem0)
        step(2 * i + 1, rows1, tail1, gsem1)
        return _

    lax.fori_loop(0, _NCH // 2, pair, 0)


@jax.jit
def _embed(ids_flat, font_table, tail_table):
    mesh = plsc.VectorSubcoreMesh(core_axis_name="c", subcore_axis_name="s")
    run = pl.kernel(
        _gather_body,
        out_type=jax.ShapeDtypeStruct((_B, EMBED_DIM), jnp.float32),
        mesh=mesh,
        scratch_types=[
            pltpu.VMEM((_BPW,), jnp.int32),
            pltpu.VMEM((_K, EMBED_DIM), jnp.float32),
            pltpu.VMEM((_K, EMBED_DIM), jnp.float32),
            pltpu.VMEM((_K, 128), jnp.float32),
            pltpu.VMEM((_K, 128), jnp.float32),
            pltpu.SemaphoreType.DMA,
            pltpu.SemaphoreType.DMA,
        ],
    )
    return run(font_table, tail_table, ids_flat)


def kernel(input_ids, font_table):
    ids_flat = input_ids.reshape(-1).astype(jnp.int32)
    tail_table = jnp.pad(font_table[:, _TAIL0:], ((0, 0), (0, 64)))
    out = _embed(ids_flat, font_table, tail_table)
    return out.reshape(BATCH, SEQ, EMBED_DIM)
